# Initial kernel scaffold; baseline (speedup 1.0000x reference)
import jax
import jax.numpy as jnp
from jax.experimental import pallas as pl


def kernel(x, vec_of_char):
    raise NotImplementedError("write your pallas kernel here")


# speedup vs baseline: 19.4721x; 19.4721x over previous
import functools

import jax
import jax.numpy as jnp
from jax import lax
from jax.experimental import pallas as pl
from jax.experimental.pallas import tpu as pltpu
from jax.experimental.pallas import tpu_sc as plsc

B, T, F = 1024, 50, 26
VOCAB, D = 1000, 16
N = B * T * F
NC, NS = 2, 16
NW = NC * NS
PER_W = N // NW
CHUNK = 2600
NCHUNK = PER_W // CHUNK


def _make_gather():
    mesh = plsc.VectorSubcoreMesh(core_axis_name="c", subcore_axis_name="s")

    @functools.partial(
        pl.kernel,
        mesh=mesh,
        out_type=jax.ShapeDtypeStruct((N, D), jnp.float32),
        scratch_types=[
            pltpu.VMEM((CHUNK,), jnp.int32),
            pltpu.VMEM((CHUNK, D), jnp.float32),
            pltpu.SemaphoreType.DMA,
        ],
        compiler_params=pltpu.CompilerParams(use_tc_tiling_on_sc=False),
    )
    def gather_kernel(idx_hbm, table_hbm, out_hbm, idx_v, rows_v, sem):
        wid = lax.axis_index("s") * NC + lax.axis_index("c")
        base = wid * PER_W

        def body(i, carry):
            off = base + i * CHUNK
            pltpu.sync_copy(idx_hbm.at[pl.ds(off, CHUNK)], idx_v)
            pltpu.async_copy(table_hbm.at[idx_v], rows_v, sem).wait()
            pltpu.sync_copy(rows_v, out_hbm.at[pl.ds(off, CHUNK)])
            return carry

        lax.fori_loop(0, NCHUNK, body, 0)

    return gather_kernel


_gather = _make_gather()


def kernel(x, vec_of_char):
    idx = x.reshape(N)
    out = _gather(idx, vec_of_char)
    return out.reshape(B, T, F * D)


# speedup vs baseline: 20.1211x; 1.0333x over previous
import functools

import jax
import jax.numpy as jnp
from jax import lax
from jax.experimental import pallas as pl
from jax.experimental.pallas import tpu as pltpu
from jax.experimental.pallas import tpu_sc as plsc

B, T, F = 1024, 50, 26
VOCAB, D = 1000, 16
N = B * T * F
NC, NS = 2, 16
NW = NC * NS
PER_W = N // NW
CHUNK = 2600
NCHUNK = PER_W // CHUNK


def _make_gather():
    mesh = plsc.VectorSubcoreMesh(core_axis_name="c", subcore_axis_name="s")

    @functools.partial(
        pl.kernel,
        mesh=mesh,
        out_type=jax.ShapeDtypeStruct((N, D), jnp.float32),
        scratch_types=[
            pltpu.VMEM((2, CHUNK), jnp.int32),
            pltpu.VMEM((2, CHUNK, D), jnp.float32),
            pltpu.SemaphoreType.DMA,
            pltpu.SemaphoreType.DMA,
            pltpu.SemaphoreType.DMA,
            pltpu.SemaphoreType.DMA,
            pltpu.SemaphoreType.DMA,
            pltpu.SemaphoreType.DMA,
        ],
        compiler_params=pltpu.CompilerParams(use_tc_tiling_on_sc=False),
    )
    def gather_kernel(idx_hbm, table_hbm, out_hbm, idx_v, rows_v,
                      isem0, isem1, gsem0, gsem1, wsem0, wsem1):
        wid = lax.axis_index("s") * NC + lax.axis_index("c")
        base = wid * PER_W
        isems = (isem0, isem1)
        gsems = (gsem0, gsem1)
        wsems = (wsem0, wsem1)

        icopies = [None, None]
        for g in range(min(2, NCHUNK)):
            icopies[g % 2] = pltpu.async_copy(
                idx_hbm.at[pl.ds(base + g * CHUNK, CHUNK)],
                idx_v.at[g % 2], isems[g % 2])

        wcopies = [None, None]
        for g in range(NCHUNK):
            b = g % 2
            off = base + g * CHUNK
            if wcopies[b] is not None:
                wcopies[b].wait()
            icopies[b].wait()
            pltpu.async_copy(table_hbm.at[idx_v.at[b]], rows_v.at[b],
                             gsems[b]).wait()
            if g + 2 < NCHUNK:
                icopies[b] = pltpu.async_copy(
                    idx_hbm.at[pl.ds(base + (g + 2) * CHUNK, CHUNK)],
                    idx_v.at[b], isems[b])
            wcopies[b] = pltpu.async_copy(
                rows_v.at[b], out_hbm.at[pl.ds(off, CHUNK)], wsems[b])

        for w in wcopies:
            if w is not None:
                w.wait()

    return gather_kernel


_gather = _make_gather()


def kernel(x, vec_of_char):
    idx = x.reshape(N)
    out = _gather(idx, vec_of_char)
    return out.reshape(B, T, F * D)
